# Initial kernel scaffold; baseline (speedup 1.0000x reference)
import jax
import jax.numpy as jnp
from jax.experimental import pallas as pl


def kernel(input, W, gamma, beta):
    raise NotImplementedError("write your pallas kernel here")


# speedup vs baseline: 342.8816x; 342.8816x over previous
import functools

import jax
import jax.numpy as jnp
from jax.experimental import pallas as pl
from jax.experimental.pallas import tpu as pltpu

_BISECT_ITERS = 22
_LEAKY = 0.01
_EPS = 1e-5


def _fused_block(x_ref, w_ref, y_ref, sum_ref, ssq_ref, s_ref):
    nb = pl.program_id(1)
    N = x_ref.shape[2]
    RB = y_ref.shape[1]
    k = int(round(N * 0.9))

    x_bf = x_ref[0].astype(jnp.bfloat16)

    @pl.when(nb == 0)
    def _():
        s_ref[...] = jax.lax.dot_general(
            x_bf, w_ref[...].astype(jnp.bfloat16),
            dimension_numbers=(((0,), (0,)), ((), ())),
            preferred_element_type=jnp.float32,
        ).astype(jnp.bfloat16)

    xb = x_ref[0, :, pl.ds(nb * RB, RB)].astype(jnp.bfloat16)
    logits = jax.lax.dot_general(
        xb, x_bf,
        dimension_numbers=(((0,), (0,)), ((), ())),
        preferred_element_type=jnp.float32)

    row_max = jnp.max(logits, axis=1, keepdims=True)
    e = jnp.exp(logits - row_max)
    denom = jnp.sum(e, axis=1, keepdims=True)
    row_min = jnp.min(logits, axis=1, keepdims=True)

    def body(_, carry):
        lo, hi = carry
        mid = 0.5 * (lo + hi)
        cnt = jnp.sum(jnp.where(logits >= mid, 1.0, 0.0), axis=1,
                      keepdims=True)
        take = cnt >= k
        return jnp.where(take, mid, lo), jnp.where(take, hi, mid)

    thr, _ = jax.lax.fori_loop(
        0, _BISECT_ITERS, body, (row_min, row_max + 1.0))

    p = jnp.where(logits >= thr, e, 0.0) * (1.0 / denom)
    out = jax.lax.dot_general(
        p.astype(jnp.bfloat16), s_ref[...],
        dimension_numbers=(((1,), (0,)), ((), ())),
        preferred_element_type=jnp.float32)
    z = jnp.where(out >= 0.0, out, _LEAKY * out)
    y_ref[0] = z
    sum_ref[0, 0] = jnp.sum(z, axis=0)
    ssq_ref[0, 0] = jnp.sum(z * z, axis=0)


def _bn_block(sum_ref, ssq_ref, g_ref, b_ref, y_ref, out_ref, *, count):
    tot = jnp.sum(sum_ref[:, 0, :], axis=0, keepdims=True)
    tot2 = jnp.sum(ssq_ref[:, 0, :], axis=0, keepdims=True)
    mean = tot / count
    var = tot2 / count - mean * mean
    inv = jax.lax.rsqrt(var + _EPS)
    scale = inv * g_ref[...]
    shift = b_ref[...] - mean * scale
    z = y_ref[0] * scale + shift
    out_ref[0] = z.T


def kernel(input, W, gamma, beta):
    B, C, N = input.shape
    O = W.shape[1]
    RB = min(512, N)
    NB = N // RB

    y, s1, s2 = pl.pallas_call(
        _fused_block,
        grid=(B, NB),
        in_specs=[
            pl.BlockSpec((1, C, N), lambda b, n: (b, 0, 0)),
            pl.BlockSpec((C, O), lambda b, n: (0, 0)),
        ],
        out_specs=[
            pl.BlockSpec((1, RB, O), lambda b, n: (b, n, 0)),
            pl.BlockSpec((1, 1, O), lambda b, n: (b * NB + n, 0, 0)),
            pl.BlockSpec((1, 1, O), lambda b, n: (b * NB + n, 0, 0)),
        ],
        out_shape=[
            jax.ShapeDtypeStruct((B, N, O), jnp.float32),
            jax.ShapeDtypeStruct((B * NB, 1, O), jnp.float32),
            jax.ShapeDtypeStruct((B * NB, 1, O), jnp.float32),
        ],
        scratch_shapes=[pltpu.VMEM((N, O), jnp.bfloat16)],
    )(input, W)

    out = pl.pallas_call(
        functools.partial(_bn_block, count=B * N),
        grid=(B, NB),
        in_specs=[
            pl.BlockSpec((B * NB, 1, O), lambda b, n: (0, 0, 0)),
            pl.BlockSpec((B * NB, 1, O), lambda b, n: (0, 0, 0)),
            pl.BlockSpec((1, O), lambda b, n: (0, 0)),
            pl.BlockSpec((1, O), lambda b, n: (0, 0)),
            pl.BlockSpec((1, RB, O), lambda b, n: (b, n, 0)),
        ],
        out_specs=pl.BlockSpec((1, O, RB), lambda b, n: (b, 0, n)),
        out_shape=jax.ShapeDtypeStruct((B, O, N), jnp.float32),
    )(s1, s2, gamma.reshape(1, O), beta.reshape(1, O), y)
    return out


# speedup vs baseline: 474.4842x; 1.3838x over previous
import functools

import jax
import jax.numpy as jnp
from jax.experimental import pallas as pl
from jax.experimental.pallas import tpu as pltpu

_BISECT_ITERS = 14
_LEAKY = 0.01
_EPS = 1e-5


def _fused_block(x_ref, w_ref, y_ref, sum_ref, ssq_ref, s_ref):
    nb = pl.program_id(1)
    N = x_ref.shape[2]
    RB = y_ref.shape[1]
    k = int(round(N * 0.9))

    x_bf = x_ref[0].astype(jnp.bfloat16)

    @pl.when(nb == 0)
    def _():
        s_ref[...] = jax.lax.dot_general(
            x_bf, w_ref[...].astype(jnp.bfloat16),
            dimension_numbers=(((0,), (0,)), ((), ())),
            preferred_element_type=jnp.float32,
        ).astype(jnp.bfloat16)

    xb = x_ref[0, :, pl.ds(nb * RB, RB)].astype(jnp.bfloat16)
    logits = jax.lax.dot_general(
        xb, x_bf,
        dimension_numbers=(((0,), (0,)), ((), ())),
        preferred_element_type=jnp.float32)

    row_max = jnp.max(logits, axis=1, keepdims=True)
    e = jnp.exp(logits - row_max)
    denom = jnp.sum(e, axis=1, keepdims=True)
    row_min = jnp.min(logits, axis=1, keepdims=True)

    def body(_, carry):
        lo, hi = carry
        mid = 0.5 * (lo + hi)
        cnt = jnp.sum(jnp.where(logits >= mid, 1.0, 0.0), axis=1,
                      keepdims=True)
        take = cnt >= k
        return jnp.where(take, mid, lo), jnp.where(take, hi, mid)

    thr, _ = jax.lax.fori_loop(
        0, _BISECT_ITERS, body, (row_min, row_max + 1.0))

    p = jnp.where(logits >= thr, e, 0.0) * (1.0 / denom)
    out = jax.lax.dot_general(
        p.astype(jnp.bfloat16), s_ref[...],
        dimension_numbers=(((1,), (0,)), ((), ())),
        preferred_element_type=jnp.float32)
    z = jnp.where(out >= 0.0, out, _LEAKY * out)
    y_ref[0] = z
    sum_ref[0, 0] = jnp.sum(z, axis=0)
    ssq_ref[0, 0] = jnp.sum(z * z, axis=0)


def _bn_block(sum_ref, ssq_ref, g_ref, b_ref, y_ref, out_ref, *, count):
    tot = jnp.sum(sum_ref[:, 0, :], axis=0, keepdims=True)
    tot2 = jnp.sum(ssq_ref[:, 0, :], axis=0, keepdims=True)
    mean = tot / count
    var = tot2 / count - mean * mean
    inv = jax.lax.rsqrt(var + _EPS)
    scale = inv * g_ref[...]
    shift = b_ref[...] - mean * scale
    z = y_ref[0] * scale + shift
    out_ref[0] = z.T


def kernel(input, W, gamma, beta):
    B, C, N = input.shape
    O = W.shape[1]
    RB = min(512, N)
    NB = N // RB

    y, s1, s2 = pl.pallas_call(
        _fused_block,
        grid=(B, NB),
        in_specs=[
            pl.BlockSpec((1, C, N), lambda b, n: (b, 0, 0)),
            pl.BlockSpec((C, O), lambda b, n: (0, 0)),
        ],
        out_specs=[
            pl.BlockSpec((1, RB, O), lambda b, n: (b, n, 0)),
            pl.BlockSpec((1, 1, O), lambda b, n: (b * NB + n, 0, 0)),
            pl.BlockSpec((1, 1, O), lambda b, n: (b * NB + n, 0, 0)),
        ],
        out_shape=[
            jax.ShapeDtypeStruct((B, N, O), jnp.float32),
            jax.ShapeDtypeStruct((B * NB, 1, O), jnp.float32),
            jax.ShapeDtypeStruct((B * NB, 1, O), jnp.float32),
        ],
        scratch_shapes=[pltpu.VMEM((N, O), jnp.bfloat16)],
    )(input, W)

    out = pl.pallas_call(
        functools.partial(_bn_block, count=B * N),
        grid=(B, NB),
        in_specs=[
            pl.BlockSpec((B * NB, 1, O), lambda b, n: (0, 0, 0)),
            pl.BlockSpec((B * NB, 1, O), lambda b, n: (0, 0, 0)),
            pl.BlockSpec((1, O), lambda b, n: (0, 0)),
            pl.BlockSpec((1, O), lambda b, n: (0, 0)),
            pl.BlockSpec((1, RB, O), lambda b, n: (b, n, 0)),
        ],
        out_specs=pl.BlockSpec((1, O, RB), lambda b, n: (b, 0, n)),
        out_shape=jax.ShapeDtypeStruct((B, O, N), jnp.float32),
    )(s1, s2, gamma.reshape(1, O), beta.reshape(1, O), y)
    return out
